# baseline (speedup 1.0000x reference)
import jax
import jax.numpy as jnp
from jax import lax
from jax.experimental import pallas as pl
from jax.experimental.pallas import tpu as pltpu
from jax.experimental.pallas import tpu_sc as plsc

B = 16384
D = 64
NC = 2
NS = 16
NW = NC * NS
BW = B // NW
IC = 128
NCH = BW // IC
LPB = D // 16
CS = BW * 16
NT = CS // IC


def _body(h_idx_hbm, t_idx_hbm, r_idx_hbm, ent_hbm, rel_hbm, perm_hbm,
          out_hbm, hi_v, ti_v, ri_v, h_v, t_v, r_v, csum_v,
          perm_v, out_v, slab, sem):
    cid = lax.axis_index("c")
    sid = lax.axis_index("s")
    wid = sid * NC + cid
    row0 = wid * NCH

    pltpu.sync_copy(h_idx_hbm.at[pl.ds(row0, NCH)], hi_v)
    pltpu.sync_copy(t_idx_hbm.at[pl.ds(row0, NCH)], ti_v)
    pltpu.sync_copy(r_idx_hbm.at[pl.ds(row0, NCH)], ri_v)
    pltpu.sync_copy(perm_hbm, perm_v)

    copies = []
    for k in range(NCH):
        copies.append(pltpu.async_copy(
            ent_hbm.at[hi_v.at[k]], h_v.at[pl.ds(k * IC, IC)], sem))
        copies.append(pltpu.async_copy(
            ent_hbm.at[ti_v.at[k]], t_v.at[pl.ds(k * IC, IC)], sem))
        copies.append(pltpu.async_copy(
            rel_hbm.at[ri_v.at[k]], r_v.at[pl.ds(k * IC, IC)], sem))
    for c in copies:
        c.wait()

    def row_body(rr, _):
        acc = None
        for c in range(LPB):
            sl = pl.ds(c * 16, 16)
            d = jnp.abs(h_v[rr, sl] + r_v[rr, sl] - t_v[rr, sl])
            acc = d if acc is None else acc + d
        csum_v[pl.ds(rr * 16, 16)] = acc
        return 0

    lax.fori_loop(0, BW, row_body, 0)

    pltpu.sync_copy(csum_v, slab.at[sid])
    tcopies = []
    for k in range(NT):
        tcopies.append(pltpu.async_copy(
            slab.at[sid].at[perm_v.at[k]],
            csum_v.at[pl.ds(k * IC, IC)], sem))
    for c in tcopies:
        c.wait()

    def grp_body(g, _):
        acc = None
        for c in range(16):
            v = csum_v[pl.ds(c * BW + g * 16, 16)]
            acc = v if acc is None else acc + v
        out_v[pl.ds(g * 16, 16)] = -acc
        return 0

    lax.fori_loop(0, BW // 16, grp_body, 0)

    pltpu.sync_copy(out_v, out_hbm.at[pl.ds(wid * BW, BW)])


@jax.jit
def _transe_sc(h_idx, t_idx, r_idx, entity_table, relation_table, perm):
    kfn = pl.kernel(
        _body,
        out_type=jax.ShapeDtypeStruct((B,), jnp.float32),
        mesh=plsc.VectorSubcoreMesh(
            core_axis_name="c", subcore_axis_name="s",
            num_cores=NC, num_subcores=NS),
        compiler_params=pltpu.CompilerParams(use_tc_tiling_on_sc=False),
        scratch_types=[
            pltpu.VMEM((NCH, IC), jnp.int32),
            pltpu.VMEM((NCH, IC), jnp.int32),
            pltpu.VMEM((NCH, IC), jnp.int32),
            pltpu.VMEM((BW, D), jnp.float32),
            pltpu.VMEM((BW, D), jnp.float32),
            pltpu.VMEM((BW, D), jnp.float32),
            pltpu.VMEM((CS,), jnp.float32),
            pltpu.VMEM((NT, IC), jnp.int32),
            pltpu.VMEM((BW,), jnp.float32),
            pltpu.VMEM_SHARED((NS, CS), jnp.float32),
            pltpu.SemaphoreType.DMA,
        ],
    )
    return kfn(h_idx, t_idx, r_idx, entity_table, relation_table, perm)


def kernel(h_idx, t_idx, r_idx, entity_table, relation_table):
    h2 = h_idx.astype(jnp.int32).reshape(B // IC, IC)
    t2 = t_idx.astype(jnp.int32).reshape(B // IC, IC)
    r2 = r_idx.astype(jnp.int32).reshape(B // IC, IC)
    k = jnp.arange(CS, dtype=jnp.int32)
    perm = ((k % BW) * 16 + k // BW).reshape(NT, IC)
    return _transe_sc(h2, t2, r2, entity_table, relation_table, perm)
